# baseline (speedup 1.0000x reference)
import functools
import jax
import jax.numpy as jnp
from jax import lax
from jax.experimental import pallas as pl
from jax.experimental.pallas import tpu as pltpu
from jax.experimental.pallas import tpu_sc as plsc

N = 10000
E = 320000
F = 128
H = 128
C = 64

NC = 2
NS = 16
NW = NC * NS
K = 128
NCHUNK = 80
EPW = K * NCHUNK
EP = NW * EPW
NP = 10240
RPT = NP // NS
DK = 80
TRASH = NP - 1


def _zero_fill(ref, nrow, ncol):
    z16 = jnp.zeros((16,), jnp.float32)

    def _f(i, c):
        for j in range(ncol // 16):
            ref[i, pl.ds(j * 16, 16)] = z16
        return c
    lax.fori_loop(0, nrow, _f, 0)


def _iota_chunk(idx_buf, base):
    lanes = lax.iota(jnp.int32, 16)
    for q in range(DK // 16):
        idx_buf[pl.ds(q * 16, 16)] = base + q * 16 + lanes


def _sc_agg_body(adj_ref, hw_ref, agg_out,
                 src2d, dst2d, idx_buf, rows, sem, agg_sh):
    cid = lax.axis_index("c")
    sid = lax.axis_index("s")
    wid = sid * NC + cid

    _zero_fill(rows, DK, F)

    for r in range(RPT // DK):
        _iota_chunk(idx_buf, sid * RPT + r * DK)
        pltpu.sync_copy(rows.at[pl.ds(0, DK)], agg_sh.at[idx_buf])

    pltpu.sync_copy(adj_ref.at[0, wid], src2d)
    pltpu.sync_copy(adj_ref.at[1, wid], dst2d)
    plsc.subcore_barrier()

    def _chunk(j, c):
        pltpu.async_copy(hw_ref.at[src2d.at[j]], rows, sem).wait()
        pltpu.sync_copy(rows, agg_sh.at[dst2d.at[j]], add=True)
        return c
    lax.fori_loop(0, NCHUNK, _chunk, 0)

    plsc.subcore_barrier()

    for r in range(RPT // DK):
        base = sid * RPT + r * DK
        _iota_chunk(idx_buf, base)
        pltpu.async_copy(agg_sh.at[idx_buf], rows.at[pl.ds(0, DK)], sem).wait()
        pltpu.sync_copy(rows.at[pl.ds(0, DK)], agg_out.at[cid, pl.ds(base, DK)])


def _sc_agg(adj, hw):
    mesh = plsc.VectorSubcoreMesh(core_axis_name="c", subcore_axis_name="s")
    run = pl.kernel(
        _sc_agg_body,
        out_type=jax.ShapeDtypeStruct((NC, NP, F), jnp.float32),
        mesh=mesh,
        scratch_types=[
            pltpu.VMEM((NCHUNK, K), jnp.int32),
            pltpu.VMEM((NCHUNK, K), jnp.int32),
            pltpu.VMEM((DK,), jnp.int32),
            pltpu.VMEM((K, F), jnp.float32),
            pltpu.SemaphoreType.DMA,
            pltpu.VMEM_SHARED((NP, F), jnp.float32),
        ],
    )
    return run(adj, hw)


def _sc_deg_body(adj_ref, deg0_out, deg1_out,
                 dst2d, idx_buf, ones, rows, sem, deg_sh):
    cid = lax.axis_index("c")
    sid = lax.axis_index("s")
    wid = sid * NC + cid

    o16 = jnp.ones((16,), jnp.float32)

    def _f(i, c):
        for j in range(F // 16):
            ones[i, pl.ds(j * 16, 16)] = o16
        return c
    lax.fori_loop(0, K, _f, 0)
    _zero_fill(rows, DK, F)

    for deg_out in (deg0_out, deg1_out):
        ell = 0 if deg_out is deg0_out else 1
        for r in range(RPT // DK):
            _iota_chunk(idx_buf, sid * RPT + r * DK)
            pltpu.sync_copy(rows, deg_sh.at[idx_buf])
        pltpu.sync_copy(adj_ref.at[ell, 1, wid], dst2d)
        plsc.subcore_barrier()

        def _chunk(j, c):
            pltpu.sync_copy(ones, deg_sh.at[dst2d.at[j]], add=True)
            return c
        lax.fori_loop(0, NCHUNK, _chunk, 0)
        plsc.subcore_barrier()

        for r in range(RPT // DK):
            base = sid * RPT + r * DK
            _iota_chunk(idx_buf, base)
            pltpu.async_copy(deg_sh.at[idx_buf], rows, sem).wait()
            pltpu.sync_copy(rows, deg_out.at[cid, pl.ds(base, DK)])
        _zero_fill(rows, DK, F)


def _sc_deg(adj):
    mesh = plsc.VectorSubcoreMesh(core_axis_name="c", subcore_axis_name="s")
    run = pl.kernel(
        _sc_deg_body,
        out_type=(
            jax.ShapeDtypeStruct((NC, NP, F), jnp.float32),
            jax.ShapeDtypeStruct((NC, NP, F), jnp.float32),
        ),
        mesh=mesh,
        scratch_types=[
            pltpu.VMEM((NCHUNK, K), jnp.int32),
            pltpu.VMEM((DK,), jnp.int32),
            pltpu.VMEM((K, F), jnp.float32),
            pltpu.VMEM((DK, F), jnp.float32),
            pltpu.SemaphoreType.DMA,
            pltpu.VMEM_SHARED((NP, F), jnp.float32),
        ],
    )
    return run(adj)


def _mm_body(x_ref, w_ref, b_ref, o_ref):
    o_ref[...] = (
        jnp.dot(x_ref[...], w_ref[...], preferred_element_type=jnp.float32)
        + b_ref[...]
    )


def _mm(x, w, b):
    n, f = x.shape
    ho = w.shape[1]
    blk = 1000
    return pl.pallas_call(
        _mm_body,
        grid=(n // blk,),
        in_specs=[
            pl.BlockSpec((blk, f), lambda i: (i, 0)),
            pl.BlockSpec((f, ho), lambda i: (0, 0)),
            pl.BlockSpec((1, ho), lambda i: (0, 0)),
        ],
        out_specs=pl.BlockSpec((blk, ho), lambda i: (i, 0)),
        out_shape=jax.ShapeDtypeStruct((n, ho), jnp.float32),
    )(x, w, b.reshape(1, ho))


def _combine_body(p_ref, d_ref, w_ref, b_ref, o_ref):
    s = p_ref[0] + p_ref[1]
    deg = jnp.maximum(d_ref[0, :, 0:1] + d_ref[1, :, 0:1], 1.0)
    t = jnp.maximum(s / deg, 0.0)
    o_ref[...] = (
        jnp.dot(t, w_ref[...], preferred_element_type=jnp.float32)
        + b_ref[...]
    )


def _combine_mm(p, d, w, b):
    ho = w.shape[1]
    blk = 1024
    return pl.pallas_call(
        _combine_body,
        grid=(NP // blk,),
        in_specs=[
            pl.BlockSpec((NC, blk, F), lambda i: (0, i, 0)),
            pl.BlockSpec((NC, blk, F), lambda i: (0, i, 0)),
            pl.BlockSpec((F, ho), lambda i: (0, 0)),
            pl.BlockSpec((1, ho), lambda i: (0, 0)),
        ],
        out_specs=pl.BlockSpec((blk, ho), lambda i: (i, 0)),
        out_shape=jax.ShapeDtypeStruct((NP, ho), jnp.float32),
    )(p, d, w, b.reshape(1, ho))


def kernel(x, adjs, W0, b0, W1, b1, W_out, b_out):
    npad = EP - E
    pad = jnp.broadcast_to(
        jnp.array([[0], [TRASH]], dtype=adjs.dtype)[None], (2, 2, npad))
    adjs_p = jnp.concatenate([adjs, pad], axis=2)
    adjs_r = adjs_p.reshape(2, 2, NW, NCHUNK, K)
    d0, d1 = _sc_deg(adjs_r)
    hw0 = _mm(x, W0, b0)
    p0 = _sc_agg(adjs_r[0], hw0)
    hw1 = _combine_mm(p0, d0, W1, b1)
    p1 = _sc_agg(adjs_r[1], hw1)
    out = _combine_mm(p1, d1, W_out, b_out)
    return out[:N]


# speedup vs baseline: 2.1180x; 2.1180x over previous
import functools
import jax
import jax.numpy as jnp
from jax import lax
from jax.experimental import pallas as pl
from jax.experimental.pallas import tpu as pltpu
from jax.experimental.pallas import tpu_sc as plsc

N = 10000
E = 320000
F = 128
H = 128
C = 64

NC = 2
NS = 16
NW = NC * NS
EPW = E // NW
K = 80
NCHUNK = EPW // K
NP = 10240
RPT = NP // NS


def _zero_fill(ref, nrow, ncol):
    z16 = jnp.zeros((16,), jnp.float32)

    def _f(i, c):
        for j in range(ncol // 16):
            ref[i, pl.ds(j * 16, 16)] = z16
        return c
    lax.fori_loop(0, nrow, _f, 0)


def _iota_chunk(idx_buf, base):
    lanes = lax.iota(jnp.int32, 16)
    for q in range(K // 16):
        idx_buf[pl.ds(q * 16, 16)] = base + q * 16 + lanes


def _sc_agg_body(adj_ref, hw_ref, agg_out,
                 src2d, dst2d, idx_buf, rows, sem, agg_sh):
    cid = lax.axis_index("c")
    sid = lax.axis_index("s")
    wid = sid * NC + cid

    _zero_fill(rows, K, F)

    for r in range(RPT // K):
        _iota_chunk(idx_buf, sid * RPT + r * K)
        pltpu.sync_copy(rows, agg_sh.at[idx_buf])

    pltpu.sync_copy(adj_ref.at[0, wid], src2d)
    pltpu.sync_copy(adj_ref.at[1, wid], dst2d)
    plsc.subcore_barrier()

    def _chunk(j, c):
        pltpu.async_copy(hw_ref.at[src2d.at[j]], rows, sem).wait()
        pltpu.sync_copy(rows, agg_sh.at[dst2d.at[j]], add=True)
        return c
    lax.fori_loop(0, NCHUNK, _chunk, 0)

    plsc.subcore_barrier()

    for r in range(RPT // K):
        base = sid * RPT + r * K
        _iota_chunk(idx_buf, base)
        pltpu.async_copy(agg_sh.at[idx_buf], rows, sem).wait()
        pltpu.sync_copy(rows, agg_out.at[cid, pl.ds(base, K)])


def _sc_agg(adj, hw):
    mesh = plsc.VectorSubcoreMesh(core_axis_name="c", subcore_axis_name="s")
    run = pl.kernel(
        _sc_agg_body,
        out_type=jax.ShapeDtypeStruct((NC, NP, F), jnp.float32),
        mesh=mesh,
        scratch_types=[
            pltpu.VMEM((NCHUNK, K), jnp.int32),
            pltpu.VMEM((NCHUNK, K), jnp.int32),
            pltpu.VMEM((K,), jnp.int32),
            pltpu.VMEM((K, F), jnp.float32),
            pltpu.SemaphoreType.DMA,
            pltpu.VMEM_SHARED((NP, F), jnp.float32),
        ],
    )
    return run(adj, hw)


def _sc_deg_body(adj_ref, deg0_out, deg1_out,
                 dst2d, idx_buf, ones, rows, sem, deg_sh):
    cid = lax.axis_index("c")
    sid = lax.axis_index("s")
    wid = sid * NC + cid

    o16 = jnp.ones((16,), jnp.float32)

    def _f(i, c):
        for j in range(F // 16):
            ones[i, pl.ds(j * 16, 16)] = o16
        return c
    lax.fori_loop(0, K, _f, 0)
    _zero_fill(rows, K, F)

    for deg_out in (deg0_out, deg1_out):
        ell = 0 if deg_out is deg0_out else 1
        for r in range(RPT // K):
            _iota_chunk(idx_buf, sid * RPT + r * K)
            pltpu.sync_copy(rows, deg_sh.at[idx_buf])
        pltpu.sync_copy(adj_ref.at[ell, 1, wid], dst2d)
        plsc.subcore_barrier()

        def _chunk(j, c):
            pltpu.sync_copy(ones, deg_sh.at[dst2d.at[j]], add=True)
            return c
        lax.fori_loop(0, NCHUNK, _chunk, 0)
        plsc.subcore_barrier()

        for r in range(RPT // K):
            base = sid * RPT + r * K
            _iota_chunk(idx_buf, base)
            pltpu.async_copy(deg_sh.at[idx_buf], rows, sem).wait()
            pltpu.sync_copy(rows, deg_out.at[cid, pl.ds(base, K)])
        _zero_fill(rows, K, F)


def _sc_deg(adj):
    mesh = plsc.VectorSubcoreMesh(core_axis_name="c", subcore_axis_name="s")
    run = pl.kernel(
        _sc_deg_body,
        out_type=(
            jax.ShapeDtypeStruct((NC, NP, F), jnp.float32),
            jax.ShapeDtypeStruct((NC, NP, F), jnp.float32),
        ),
        mesh=mesh,
        scratch_types=[
            pltpu.VMEM((NCHUNK, K), jnp.int32),
            pltpu.VMEM((K,), jnp.int32),
            pltpu.VMEM((K, F), jnp.float32),
            pltpu.VMEM((K, F), jnp.float32),
            pltpu.SemaphoreType.DMA,
            pltpu.VMEM_SHARED((NP, F), jnp.float32),
        ],
    )
    return run(adj)


def _mm_body(x_ref, w_ref, b_ref, o_ref):
    o_ref[...] = (
        jnp.dot(x_ref[...], w_ref[...], preferred_element_type=jnp.float32)
        + b_ref[...]
    )


def _mm(x, w, b):
    n, f = x.shape
    ho = w.shape[1]
    blk = 1000
    return pl.pallas_call(
        _mm_body,
        grid=(n // blk,),
        in_specs=[
            pl.BlockSpec((blk, f), lambda i: (i, 0)),
            pl.BlockSpec((f, ho), lambda i: (0, 0)),
            pl.BlockSpec((1, ho), lambda i: (0, 0)),
        ],
        out_specs=pl.BlockSpec((blk, ho), lambda i: (i, 0)),
        out_shape=jax.ShapeDtypeStruct((n, ho), jnp.float32),
    )(x, w, b.reshape(1, ho))


def _combine_body(p_ref, d_ref, w_ref, b_ref, o_ref):
    s = p_ref[0] + p_ref[1]
    deg = jnp.maximum(d_ref[0, :, 0:1] + d_ref[1, :, 0:1], 1.0)
    t = jnp.maximum(s / deg, 0.0)
    o_ref[...] = (
        jnp.dot(t, w_ref[...], preferred_element_type=jnp.float32)
        + b_ref[...]
    )


def _combine_mm(p, d, w, b):
    ho = w.shape[1]
    blk = 1024
    return pl.pallas_call(
        _combine_body,
        grid=(NP // blk,),
        in_specs=[
            pl.BlockSpec((NC, blk, F), lambda i: (0, i, 0)),
            pl.BlockSpec((NC, blk, F), lambda i: (0, i, 0)),
            pl.BlockSpec((F, ho), lambda i: (0, 0)),
            pl.BlockSpec((1, ho), lambda i: (0, 0)),
        ],
        out_specs=pl.BlockSpec((blk, ho), lambda i: (i, 0)),
        out_shape=jax.ShapeDtypeStruct((NP, ho), jnp.float32),
    )(p, d, w, b.reshape(1, ho))


def kernel(x, adjs, W0, b0, W1, b1, W_out, b_out):
    adjs_r = adjs.reshape(2, 2, NW, NCHUNK, K)
    d0, d1 = _sc_deg(adjs_r)
    hw0 = _mm(x, W0, b0)
    p0 = _sc_agg(adjs_r[0], hw0)
    hw1 = _combine_mm(p0, d0, W1, b1)
    p1 = _sc_agg(adjs_r[1], hw1)
    out = _combine_mm(p1, d1, W_out, b_out)
    return out[:N]
